# baseline (speedup 1.0000x reference)
import jax
import jax.numpy as jnp
from jax.experimental import pallas as pl
from jax.experimental.pallas import tpu as pltpu

_G = 64
_V = 64
_C = 8
_K = 64

_BT = 256
_NBUF = 6


def _scatter_w2_kernel(w_ref, ind_ref, w2_ref):
    w = w_ref[...]
    ind = ind_ref[...]
    kk = jax.lax.broadcasted_iota(jnp.int32, (_K, _G), 0)
    acc = jnp.zeros((_K, _G, _V), jnp.float32)
    for c in range(_C):
        mask = (ind[:, c][None, :] == kk).astype(jnp.float32)
        acc = acc + mask[:, :, None] * w[:, :, c][None, :, :]
    w2_ref[...] = acc


def _matmul_kernel(nsteps, x_ref, w2_ref, out_ref, buf_ref, sem_ref):
    i = pl.program_id(0)
    slot = jax.lax.rem(i, _NBUF)

    def _dst(step):
        return out_ref.at[pl.ds(step * _BT, _BT)]

    @pl.when(i >= _NBUF)
    def _wait_prev():
        pltpu.make_async_copy(
            buf_ref.at[slot], _dst(i - _NBUF), sem_ref.at[slot]
        ).wait()

    res = jnp.dot(x_ref[...], w2_ref[...], preferred_element_type=jnp.float32)
    buf_ref[slot] = res.reshape(_BT, _G, _V)
    pltpu.make_async_copy(
        buf_ref.at[slot], _dst(i), sem_ref.at[slot]
    ).start()

    @pl.when(i == nsteps - 1)
    def _drain():
        for j in range(_NBUF):
            step = nsteps - _NBUF + j
            s = step % _NBUF
            pltpu.make_async_copy(
                buf_ref.at[s], _dst(step), sem_ref.at[s]
            ).wait()


@jax.jit
def kernel(x, w, ind):
    B = x.shape[0]
    nsteps = B // _BT

    w2 = pl.pallas_call(
        _scatter_w2_kernel,
        out_shape=jax.ShapeDtypeStruct((_K, _G, _V), jnp.float32),
    )(w, ind)
    w2 = w2.reshape(_K, _G * _V)

    out = pl.pallas_call(
        lambda *refs: _matmul_kernel(nsteps, *refs),
        grid=(nsteps,),
        in_specs=[
            pl.BlockSpec((_BT, _K), lambda i: (i, 0)),
            pl.BlockSpec((_K, _G * _V), lambda i: (0, 0)),
        ],
        out_specs=pl.BlockSpec(memory_space=pl.MemorySpace.ANY),
        out_shape=jax.ShapeDtypeStruct((B, _G, _V), jnp.float32),
        scratch_shapes=[
            pltpu.VMEM((_NBUF, _BT, _G, _V), jnp.float32),
            pltpu.SemaphoreType.DMA((_NBUF,)),
        ],
    )(x, w2)
    return out


# speedup vs baseline: 6.0531x; 6.0531x over previous
import jax
import jax.numpy as jnp
from jax.experimental import pallas as pl
from jax.experimental.pallas import tpu as pltpu

_G = 64
_V = 64
_C = 8
_K = 64
_RB = 128


def _fused_kernel(w_ref, ind_ref, xt_ref, out_ref, w2t_ref):
    i = pl.program_id(0)

    @pl.when(i == 0)
    def _build_w2t():
        w = w_ref[...]
        ind = ind_ref[...]
        kk = jax.lax.broadcasted_iota(jnp.int32, (_G, _C, _K), 2)
        m = (ind[:, :, None] == kk).astype(jnp.float32)
        w2t = jax.lax.dot_general(
            w, m, (((2,), (1,)), ((0,), (0,))),
            preferred_element_type=jnp.float32,
        )
        w2t_ref[...] = w2t.reshape(_G * _V, _K)

    out_ref[...] = jnp.dot(
        w2t_ref[pl.ds(i * _RB, _RB), :], xt_ref[...],
        preferred_element_type=jnp.float32,
    )


@jax.jit
def kernel(x, w, ind):
    B = x.shape[0]
    xt = x.T

    outt = pl.pallas_call(
        _fused_kernel,
        grid=(_G * _V // _RB,),
        in_specs=[
            pl.BlockSpec((_G, _V, _C), lambda i: (0, 0, 0)),
            pl.BlockSpec((_G, _C), lambda i: (0, 0)),
            pl.BlockSpec((_K, B), lambda i: (0, 0)),
        ],
        out_specs=pl.BlockSpec((_RB, B), lambda i: (i, 0)),
        out_shape=jax.ShapeDtypeStruct((_G * _V, B), jnp.float32),
        scratch_shapes=[pltpu.VMEM((_G * _V, _K), jnp.float32)],
    )(w, ind, xt)
    return outt.reshape(_G, _V, B).transpose(2, 0, 1)
